# baseline (device time: 119559 ns/iter reference)
import jax
import jax.numpy as jnp
from jax import lax
from jax.experimental import pallas as pl
from jax.experimental.pallas import tpu as pltpu

N_DEV = 4
SQ = 1024
SKV = 1024
HQ_PER = 8
DH = 128
BLK = 64
SCALE = 0.08838834764831843


def _attn_body(x_ref, wq_ref, k_ref, v_ref, wo_ref, out_ref, ctx_ref):
    q = jnp.dot(x_ref[...], wq_ref[...], preferred_element_type=jnp.float32)
    q = (q * SCALE).astype(jnp.bfloat16)

    row = lax.broadcasted_iota(jnp.int32, (SQ, SKV), 0) // BLK
    col = lax.broadcasted_iota(jnp.int32, (SQ, SKV), 1) // BLK
    mask = (row == col) | (col == 0) | (((row + col) % 3) == 0)

    for h in range(HQ_PER):
        q_h = q[:, h * DH:(h + 1) * DH]
        k_h = k_ref[:, h, :]
        s = lax.dot_general(
            q_h, k_h, (((1,), (1,)), ((), ())),
            preferred_element_type=jnp.float32,
        )
        s = jnp.where(mask, s, -1e9)
        m = jnp.max(s, axis=1, keepdims=True)
        w = jnp.exp(s - m)
        w = w / jnp.sum(w, axis=1, keepdims=True)
        ctx_h = jnp.dot(
            w.astype(jnp.bfloat16), v_ref[:, h, :],
            preferred_element_type=jnp.float32,
        )
        ctx_ref[:, h * DH:(h + 1) * DH] = ctx_h.astype(jnp.bfloat16)

    out_ref[...] = jnp.dot(
        ctx_ref[...], wo_ref[...], preferred_element_type=jnp.float32
    )


def _allreduce_body(p_ref, out_ref, comm_ref, send_sems, recv_sems):
    my = lax.axis_index("i")
    left = lax.rem(my - 1 + N_DEV, N_DEV)
    right = lax.rem(my + 1, N_DEV)

    barrier_sem = pltpu.get_barrier_semaphore()
    for nbr in (left, right):
        pl.semaphore_signal(
            barrier_sem, inc=1,
            device_id=(nbr,), device_id_type=pl.DeviceIdType.MESH,
        )
    pl.semaphore_wait(barrier_sem, 2)

    comm_ref[0, :, :] = p_ref[...].astype(jnp.bfloat16)
    acc = p_ref[...]

    for h in range(N_DEV - 1):
        rdma = pltpu.make_async_remote_copy(
            src_ref=comm_ref.at[h],
            dst_ref=comm_ref.at[h + 1],
            send_sem=send_sems.at[h],
            recv_sem=recv_sems.at[h],
            device_id=(right,),
            device_id_type=pl.DeviceIdType.MESH,
        )
        rdma.start()
        rdma.wait()
        acc = acc + comm_ref[h + 1, :, :].astype(jnp.float32)

    out_ref[0, :, :] = acc


def kernel(x, Wq, K_ext, V_ext, Wo):
    my = lax.axis_index("i")

    x2 = x[0].astype(jnp.bfloat16)
    wq = Wq.astype(jnp.bfloat16)
    wo = Wo.astype(jnp.bfloat16)
    k = lax.dynamic_slice_in_dim(K_ext[0], my * HQ_PER, HQ_PER, axis=1)
    v = lax.dynamic_slice_in_dim(V_ext[0], my * HQ_PER, HQ_PER, axis=1)
    k = k.astype(jnp.bfloat16)
    v = v.astype(jnp.bfloat16)

    partial = pl.pallas_call(
        _attn_body,
        out_shape=jax.ShapeDtypeStruct((SQ, 1024), jnp.float32),
        in_specs=[pl.BlockSpec(memory_space=pltpu.VMEM)] * 5,
        out_specs=pl.BlockSpec(memory_space=pltpu.VMEM),
        scratch_shapes=[pltpu.VMEM((SQ, HQ_PER * DH), jnp.bfloat16)],
    )(x2, wq, k, v, wo)

    return pl.pallas_call(
        _allreduce_body,
        out_shape=jax.ShapeDtypeStruct((1, SQ, 1024), jnp.float32),
        in_specs=[pl.BlockSpec(memory_space=pltpu.VMEM)],
        out_specs=pl.BlockSpec(memory_space=pltpu.VMEM),
        scratch_shapes=[
            pltpu.VMEM((N_DEV, SQ, 1024), jnp.bfloat16),
            pltpu.SemaphoreType.DMA((N_DEV - 1,)),
            pltpu.SemaphoreType.DMA((N_DEV - 1,)),
        ],
        compiler_params=pltpu.CompilerParams(collective_id=0),
    )(partial)


# device time: 85161 ns/iter; 1.4039x vs baseline; 1.4039x over previous
import jax
import jax.numpy as jnp
from jax import lax
from jax.experimental import pallas as pl
from jax.experimental.pallas import tpu as pltpu

N_DEV = 4
SQ = 1024
SKV = 1024
HQ_PER = 8
DH = 128
BLK = 64
CHUNK = SQ // N_DEV
SCALE = 0.08838834764831843


def _fused_body(
    x_ref, wq_ref, k_ref, v_ref, wo_ref, out_ref,
    q_scr, rs_send, rs_recv, ag_buf,
    rs_ssem, rs_rsem, ag_ssem, ag_rsem,
):
    my = lax.axis_index("i")
    left = lax.rem(my + N_DEV - 1, N_DEV)
    right = lax.rem(my + 1, N_DEV)

    barrier_sem = pltpu.get_barrier_semaphore()
    for nbr in (left, right):
        pl.semaphore_signal(
            barrier_sem, inc=1,
            device_id=(nbr,), device_id_type=pl.DeviceIdType.MESH,
        )
    pl.semaphore_wait(barrier_sem, 2)

    kb = lax.broadcasted_iota(jnp.int32, (CHUNK, SKV), 1) // BLK

    def compute_chunk(c):
        xs = x_ref[pl.ds(c * CHUNK, CHUNK), :]
        q = jnp.dot(xs, wq_ref[...], preferred_element_type=jnp.float32)
        q_scr[...] = (q * SCALE).astype(jnp.bfloat16)
        rowi = c * CHUNK + lax.broadcasted_iota(jnp.int32, (CHUNK, SKV), 0)
        qb = rowi // BLK
        mask = (qb == kb) | (kb == 0) | (((qb + kb) % 3) == 0)

        def head(h, p):
            q_h = q_scr[:, pl.ds(h * DH, DH)]
            s = lax.dot_general(
                q_h, k_ref[h], (((1,), (1,)), ((), ())),
                preferred_element_type=jnp.float32,
            )
            s = jnp.where(mask, s, -1e9)
            mx = jnp.max(s, axis=1, keepdims=True)
            w = jnp.exp(s - mx)
            w = w / jnp.sum(w, axis=1, keepdims=True)
            ctx_h = jnp.dot(
                w.astype(jnp.bfloat16), v_ref[h],
                preferred_element_type=jnp.float32,
            ).astype(jnp.bfloat16)
            return p + jnp.dot(
                ctx_h, wo_ref[pl.ds(h * DH, DH), :],
                preferred_element_type=jnp.float32,
            )

        return lax.fori_loop(
            0, HQ_PER, head, jnp.zeros((CHUNK, 1024), jnp.float32)
        )

    def rs_rdma(s):
        return pltpu.make_async_remote_copy(
            src_ref=rs_send.at[s], dst_ref=rs_recv.at[s],
            send_sem=rs_ssem.at[s], recv_sem=rs_rsem.at[s],
            device_id=(right,), device_id_type=pl.DeviceIdType.MESH,
        )

    rs_send[0, :, :] = compute_chunk(my).astype(jnp.bfloat16)
    prev = rs_rdma(0)
    prev.start()
    for s in (1, 2):
        c = lax.rem(my - s + N_DEV, N_DEV)
        local = compute_chunk(c)
        prev.wait()
        rs_send[s, :, :] = (
            local + rs_recv[s - 1, :, :].astype(jnp.float32)
        ).astype(jnp.bfloat16)
        prev = rs_rdma(s)
        prev.start()
    c_own = lax.rem(my + 1, N_DEV)
    local = compute_chunk(c_own)
    prev.wait()
    owned = local + rs_recv[2, :, :].astype(jnp.float32)
    out_ref[0, pl.ds(c_own * CHUNK, CHUNK), :] = owned

    ag_buf[0, :, :] = owned.astype(jnp.bfloat16)
    ags = [
        pltpu.make_async_remote_copy(
            src_ref=ag_buf.at[h], dst_ref=ag_buf.at[h + 1],
            send_sem=ag_ssem.at[h], recv_sem=ag_rsem.at[h],
            device_id=(right,), device_id_type=pl.DeviceIdType.MESH,
        )
        for h in range(N_DEV - 1)
    ]
    ags[0].start()
    for h in range(N_DEV - 1):
        ags[h].wait()
        if h < N_DEV - 2:
            ags[h + 1].start()
        oc = lax.rem(my - h + N_DEV, N_DEV)
        out_ref[0, pl.ds(oc * CHUNK, CHUNK), :] = (
            ag_buf[h + 1, :, :].astype(jnp.float32)
        )


def kernel(x, Wq, K_ext, V_ext, Wo):
    my = lax.axis_index("i")

    x2 = x[0].astype(jnp.bfloat16)
    wq = Wq.astype(jnp.bfloat16)
    wo = Wo.astype(jnp.bfloat16)
    k = lax.dynamic_slice_in_dim(K_ext[0], my * HQ_PER, HQ_PER, axis=1)
    v = lax.dynamic_slice_in_dim(V_ext[0], my * HQ_PER, HQ_PER, axis=1)
    k = jnp.transpose(k, (1, 0, 2)).astype(jnp.bfloat16)
    v = jnp.transpose(v, (1, 0, 2)).astype(jnp.bfloat16)

    return pl.pallas_call(
        _fused_body,
        out_shape=jax.ShapeDtypeStruct((1, SQ, 1024), jnp.float32),
        in_specs=[pl.BlockSpec(memory_space=pltpu.VMEM)] * 5,
        out_specs=pl.BlockSpec(memory_space=pltpu.VMEM),
        scratch_shapes=[
            pltpu.VMEM((CHUNK, HQ_PER * DH), jnp.bfloat16),
            pltpu.VMEM((N_DEV - 1, CHUNK, 1024), jnp.bfloat16),
            pltpu.VMEM((N_DEV - 1, CHUNK, 1024), jnp.bfloat16),
            pltpu.VMEM((N_DEV, CHUNK, 1024), jnp.bfloat16),
            pltpu.SemaphoreType.DMA((N_DEV - 1,)),
            pltpu.SemaphoreType.DMA((N_DEV - 1,)),
            pltpu.SemaphoreType.DMA((N_DEV - 1,)),
            pltpu.SemaphoreType.DMA((N_DEV - 1,)),
        ],
        compiler_params=pltpu.CompilerParams(collective_id=0),
    )(x2, wq, k, v, wo)


# device time: 67340 ns/iter; 1.7755x vs baseline; 1.2646x over previous
import jax
import jax.numpy as jnp
from jax import lax
from jax.experimental import pallas as pl
from jax.experimental.pallas import tpu as pltpu

N_DEV = 4
SQ = 1024
SKV = 1024
HQ_PER = 8
DH = 128
BLK = 64
CHUNK = SQ // N_DEV
SCALE = 0.08838834764831843
NEG = -1e9


def _fused_body(
    x_ref, wq_ref, k_ref, v_ref, wo_ref, out_ref,
    q_scr, rs_send, rs_recv, ag_send, ag_recv,
    rs_ssem, rs_rsem, ag_ssem, ag_rsem,
):
    my = lax.axis_index("i")

    barrier_sem = pltpu.get_barrier_semaphore()
    for j in (1, 2, 3):
        pl.semaphore_signal(
            barrier_sem, inc=1,
            device_id=(lax.rem(my + j, N_DEV),),
            device_id_type=pl.DeviceIdType.MESH,
        )
    pl.semaphore_wait(barrier_sem, 3)

    kb = lax.broadcasted_iota(jnp.int32, (CHUNK, SKV), 1) // BLK

    def compute_chunk(c):
        xs = x_ref[pl.ds(c * CHUNK, CHUNK), :]
        q = jnp.dot(xs, wq_ref[...], preferred_element_type=jnp.float32)
        q_scr[...] = (q * SCALE).astype(jnp.bfloat16)
        rowi = c * CHUNK + lax.broadcasted_iota(jnp.int32, (CHUNK, SKV), 0)
        qb = rowi // BLK
        keep = (qb == kb) | (kb == 0) | (((qb + kb) % 3) == 0)
        bias = jnp.where(keep, 0.0, NEG).astype(jnp.float32)

        def head(h, p):
            q_h = q_scr[:, pl.ds(h * DH, DH)]
            s = lax.dot_general(
                q_h, k_ref[h], (((1,), (1,)), ((), ())),
                preferred_element_type=jnp.float32,
            ) + bias
            w = jnp.exp(s)
            ws = jnp.sum(w, axis=1, keepdims=True)
            ctx_h = jnp.dot(
                w.astype(jnp.bfloat16), v_ref[h],
                preferred_element_type=jnp.float32,
            )
            ctx_h = (ctx_h * (1.0 / ws)).astype(jnp.bfloat16)
            return p + jnp.dot(
                ctx_h, wo_ref[pl.ds(h * DH, DH), :],
                preferred_element_type=jnp.float32,
            )

        return lax.fori_loop(
            0, HQ_PER, head, jnp.zeros((CHUNK, 1024), jnp.float32)
        )

    rs = []
    for j in (1, 2, 3):
        tgt = lax.rem(my + j, N_DEV)
        rs_send[j - 1, :, :] = compute_chunk(tgt).astype(jnp.bfloat16)
        r = pltpu.make_async_remote_copy(
            src_ref=rs_send.at[j - 1], dst_ref=rs_recv.at[j - 1],
            send_sem=rs_ssem.at[j - 1], recv_sem=rs_rsem.at[j - 1],
            device_id=(tgt,), device_id_type=pl.DeviceIdType.MESH,
        )
        r.start()
        rs.append(r)

    acc = compute_chunk(my)
    for j in (1, 2, 3):
        rs[j - 1].wait()
        acc = acc + rs_recv[j - 1, :, :].astype(jnp.float32)
    out_ref[0, pl.ds(my * CHUNK, CHUNK), :] = acc

    ag_send[...] = acc.astype(jnp.bfloat16)
    ag = []
    for j in (1, 2, 3):
        tgt = lax.rem(my + j, N_DEV)
        a = pltpu.make_async_remote_copy(
            src_ref=ag_send, dst_ref=ag_recv.at[j - 1],
            send_sem=ag_ssem.at[j - 1], recv_sem=ag_rsem.at[j - 1],
            device_id=(tgt,), device_id_type=pl.DeviceIdType.MESH,
        )
        a.start()
        ag.append(a)
    for j in (1, 2, 3):
        ag[j - 1].wait()
        oc = lax.rem(my - j + N_DEV, N_DEV)
        out_ref[0, pl.ds(oc * CHUNK, CHUNK), :] = (
            ag_recv[j - 1, :, :].astype(jnp.float32)
        )


def kernel(x, Wq, K_ext, V_ext, Wo):
    my = lax.axis_index("i")

    x2 = x[0].astype(jnp.bfloat16)
    wq = Wq.astype(jnp.bfloat16)
    wo = Wo.astype(jnp.bfloat16)
    k = lax.dynamic_slice_in_dim(K_ext[0], my * HQ_PER, HQ_PER, axis=1)
    v = lax.dynamic_slice_in_dim(V_ext[0], my * HQ_PER, HQ_PER, axis=1)
    k = jnp.transpose(k, (1, 0, 2)).astype(jnp.bfloat16)
    v = jnp.transpose(v, (1, 0, 2)).astype(jnp.bfloat16)

    return pl.pallas_call(
        _fused_body,
        out_shape=jax.ShapeDtypeStruct((1, SQ, 1024), jnp.float32),
        in_specs=[pl.BlockSpec(memory_space=pltpu.VMEM)] * 5,
        out_specs=pl.BlockSpec(memory_space=pltpu.VMEM),
        scratch_shapes=[
            pltpu.VMEM((CHUNK, HQ_PER * DH), jnp.bfloat16),
            pltpu.VMEM((N_DEV - 1, CHUNK, 1024), jnp.bfloat16),
            pltpu.VMEM((N_DEV - 1, CHUNK, 1024), jnp.bfloat16),
            pltpu.VMEM((CHUNK, 1024), jnp.bfloat16),
            pltpu.VMEM((N_DEV - 1, CHUNK, 1024), jnp.bfloat16),
            pltpu.SemaphoreType.DMA((N_DEV - 1,)),
            pltpu.SemaphoreType.DMA((N_DEV - 1,)),
            pltpu.SemaphoreType.DMA((N_DEV - 1,)),
            pltpu.SemaphoreType.DMA((N_DEV - 1,)),
        ],
        compiler_params=pltpu.CompilerParams(collective_id=0),
    )(x2, wq, k, v, wo)


# device time: 65631 ns/iter; 1.8217x vs baseline; 1.0260x over previous
import jax
import jax.numpy as jnp
from jax import lax
from jax.experimental import pallas as pl
from jax.experimental.pallas import tpu as pltpu

N_DEV = 4
SQ = 1024
SKV = 1024
HQ_PER = 8
DH = 128
BLK = 64
CHUNK = SQ // N_DEV
SCALE = 0.08838834764831843
NEG = -1e9


HALF = CHUNK // 2


def _fused_body(
    x_ref, wq_ref, k_ref, v_ref, wo_ref, out_ref,
    q_scr, rs_send, rs_recv, ag_send0, ag_send1, ag_recv,
    rs_ssem, rs_rsem, ag_ssem, ag_rsem,
):
    my = lax.axis_index("i")

    barrier_sem = pltpu.get_barrier_semaphore()
    for j in (1, 2, 3):
        pl.semaphore_signal(
            barrier_sem, inc=1,
            device_id=(lax.rem(my + j, N_DEV),),
            device_id_type=pl.DeviceIdType.MESH,
        )
    pl.semaphore_wait(barrier_sem, 3)

    def compute_rows(start, nrows):
        xs = x_ref[pl.ds(start, nrows), :]
        q = jnp.dot(xs, wq_ref[...], preferred_element_type=jnp.float32)
        q_scr[pl.ds(0, nrows), :] = (q * SCALE).astype(jnp.bfloat16)
        rowi = start + lax.broadcasted_iota(jnp.int32, (nrows, SKV), 0)
        qb = rowi // BLK
        kb = lax.broadcasted_iota(jnp.int32, (nrows, SKV), 1) // BLK
        keep = (qb == kb) | (kb == 0) | (((qb + kb) % 3) == 0)
        bias = jnp.where(keep, 0.0, NEG).astype(jnp.float32)

        def head(h, p):
            q_h = q_scr[pl.ds(0, nrows), pl.ds(h * DH, DH)]
            s = lax.dot_general(
                q_h, k_ref[h], (((1,), (1,)), ((), ())),
                preferred_element_type=jnp.float32,
            ) + bias
            w = jnp.exp(s)
            ws = jnp.sum(w, axis=1, keepdims=True)
            ctx_h = jnp.dot(
                w.astype(jnp.bfloat16), v_ref[h],
                preferred_element_type=jnp.float32,
            )
            ctx_h = (ctx_h * (1.0 / ws)).astype(jnp.bfloat16)
            return p + jnp.dot(
                ctx_h, wo_ref[pl.ds(h * DH, DH), :],
                preferred_element_type=jnp.float32,
            )

        return lax.fori_loop(
            0, HQ_PER, head, jnp.zeros((nrows, 1024), jnp.float32)
        )

    rs = []
    for j in (1, 2, 3):
        tgt = lax.rem(my + j, N_DEV)
        rs_send[j - 1, :, :] = (
            compute_rows(tgt * CHUNK, CHUNK).astype(jnp.bfloat16)
        )
        r = pltpu.make_async_remote_copy(
            src_ref=rs_send.at[j - 1], dst_ref=rs_recv.at[j - 1],
            send_sem=rs_ssem.at[j - 1], recv_sem=rs_rsem.at[j - 1],
            device_id=(tgt,), device_id_type=pl.DeviceIdType.MESH,
        )
        r.start()
        rs.append(r)

    ag = []
    ag_srcs = (ag_send0, ag_send1)
    for half in (0, 1):
        base = my * CHUNK + half * HALF
        acc = compute_rows(base, HALF)
        if half == 0:
            for j in (1, 2, 3):
                rs[j - 1].wait()
        for j in (1, 2, 3):
            acc = acc + rs_recv[
                j - 1, pl.ds(half * HALF, HALF), :
            ].astype(jnp.float32)
        out_ref[0, pl.ds(base, HALF), :] = acc
        ag_srcs[half][...] = acc.astype(jnp.bfloat16)
        for j in (1, 2, 3):
            tgt = lax.rem(my + j, N_DEV)
            a = pltpu.make_async_remote_copy(
                src_ref=ag_srcs[half],
                dst_ref=ag_recv.at[j - 1, half],
                send_sem=ag_ssem.at[j - 1, half],
                recv_sem=ag_rsem.at[j - 1, half],
                device_id=(tgt,), device_id_type=pl.DeviceIdType.MESH,
            )
            a.start()
            ag.append(a)

    for i, a in enumerate(ag):
        a.wait()
        j = i % 3 + 1
        half = i // 3
        oc = lax.rem(my - j + N_DEV, N_DEV)
        out_ref[0, pl.ds(oc * CHUNK + half * HALF, HALF), :] = (
            ag_recv[j - 1, half, :, :].astype(jnp.float32)
        )


def kernel(x, Wq, K_ext, V_ext, Wo):
    my = lax.axis_index("i")

    x2 = x[0].astype(jnp.bfloat16)
    wq = Wq.astype(jnp.bfloat16)
    wo = Wo.astype(jnp.bfloat16)
    k = lax.dynamic_slice_in_dim(K_ext[0], my * HQ_PER, HQ_PER, axis=1)
    v = lax.dynamic_slice_in_dim(V_ext[0], my * HQ_PER, HQ_PER, axis=1)
    k = jnp.transpose(k, (1, 0, 2)).astype(jnp.bfloat16)
    v = jnp.transpose(v, (1, 0, 2)).astype(jnp.bfloat16)

    return pl.pallas_call(
        _fused_body,
        out_shape=jax.ShapeDtypeStruct((1, SQ, 1024), jnp.float32),
        in_specs=[pl.BlockSpec(memory_space=pltpu.VMEM)] * 5,
        out_specs=pl.BlockSpec(memory_space=pltpu.VMEM),
        scratch_shapes=[
            pltpu.VMEM((CHUNK, HQ_PER * DH), jnp.bfloat16),
            pltpu.VMEM((N_DEV - 1, CHUNK, 1024), jnp.bfloat16),
            pltpu.VMEM((N_DEV - 1, CHUNK, 1024), jnp.bfloat16),
            pltpu.VMEM((HALF, 1024), jnp.bfloat16),
            pltpu.VMEM((HALF, 1024), jnp.bfloat16),
            pltpu.VMEM((N_DEV - 1, 2, HALF, 1024), jnp.bfloat16),
            pltpu.SemaphoreType.DMA((N_DEV - 1,)),
            pltpu.SemaphoreType.DMA((N_DEV - 1,)),
            pltpu.SemaphoreType.DMA((N_DEV - 1, 2)),
            pltpu.SemaphoreType.DMA((N_DEV - 1, 2)),
        ],
        compiler_params=pltpu.CompilerParams(collective_id=0),
    )(x2, wq, k, v, wo)
